# baseline (speedup 1.0000x reference)
import functools

import jax
import jax.numpy as jnp
from jax import lax
from jax.experimental import pallas as pl
from jax.experimental.pallas import tpu as pltpu
from jax.experimental.pallas import tpu_sc as plsc

N_IMG, N_CLS, H, W = 4, 19, 512, 512
N_PIX = N_IMG * H * W
K_HARD = max(100000, int(N_PIX * 0.7))
HB = 64
N_HB = H // HB
BISECT_ITERS = 15

NW = 32
PER_W = N_PIX // NW
LANE_BINS = 32
BINS_W = 16 * LANE_BINS


def _classhist_sc_body(t_hbm, out_hbm, t_v, bins_v):
    wid = lax.axis_index("s") * 2 + lax.axis_index("c")
    base = wid * PER_W
    pltpu.sync_copy(t_hbm.at[pl.ds(base, PER_W)], t_v)
    zero = jnp.zeros((16,), jnp.float32)
    for j in range(BINS_W // 16):
        bins_v[pl.ds(j * 16, 16)] = zero
    lanes = lax.iota(jnp.int32, 16) * LANE_BINS
    ones = jnp.ones((16,), jnp.float32)

    def body(i, carry):
        t16 = t_v[pl.ds(pl.multiple_of(i * 16, 16), 16)]
        plsc.addupdate_scatter(bins_v, [lanes + t16], ones)
        return carry

    lax.fori_loop(0, PER_W // 16, body, 0)
    pltpu.sync_copy(bins_v, out_hbm.at[wid])


def _classhist_sc(tflat):
    return pl.kernel(
        _classhist_sc_body,
        out_type=jax.ShapeDtypeStruct((NW, BINS_W), jnp.float32),
        mesh=plsc.VectorSubcoreMesh(core_axis_name="c", subcore_axis_name="s"),
        scratch_types=[
            pltpu.VMEM((PER_W,), jnp.int32),
            pltpu.VMEM((BINS_W,), jnp.float32),
        ],
        compiler_params=pltpu.CompilerParams(needs_layout_passes=False),
    )(tflat)


def _ohem_body(p_ref, t_ref, hp_ref, out_ref, loss_buf, w_sm):
    n = pl.program_id(0)
    h = pl.program_id(1)

    @pl.when((n == 0) & (h == 0))
    def _():
        cnts = jnp.sum(hp_ref[...], axis=0)
        for c in range(N_CLS):
            w_sm[c] = 2.0 - cnts[c] * (1.0 / N_PIX)

    p = p_ref[0]
    t = t_ref[...]
    s = jnp.zeros((HB, W), jnp.float32)
    pt = jnp.zeros((HB, W), jnp.float32)
    wp = jnp.zeros((HB, W), jnp.float32)
    for c in range(N_CLS):
        pc = p[c]
        s = s + jnp.exp(pc)
        m = t == c
        pt = pt + jnp.where(m, pc, 0.0)
        wp = wp + jnp.where(m, w_sm[c], 0.0)
    loss = wp * (jnp.log(s) - pt)
    row = (n * N_HB + h) * HB
    loss_buf[pl.ds(row, HB), :] = loss

    @pl.when((n == N_IMG - 1) & (h == N_HB - 1))
    def _():
        lb = loss_buf[...]
        kf = jnp.float32(K_HARD)

        def it(_, carry):
            lo, hi = carry
            mid = 0.5 * (lo + hi)
            cnt = jnp.sum((lb > mid).astype(jnp.float32))
            take = cnt >= kf
            return jnp.where(take, mid, lo), jnp.where(take, hi, mid)

        lo, hi = jax.lax.fori_loop(
            0, BISECT_ITERS, it, (jnp.float32(0.0), jnp.max(lb)))
        mid = 0.5 * (lo + hi)
        msk = lb > hi
        cnt_gt = jnp.sum(msk.astype(jnp.float32))
        sum_gt = jnp.sum(jnp.where(msk, lb, 0.0))
        hard_sum = sum_gt + (kf - cnt_gt) * mid
        loss_val = hard_sum * (1.0 / (H * W)) * (1.0 / N_IMG)
        out_ref[...] = jnp.full((1, 1), loss_val, jnp.float32)


@functools.partial(jax.jit, static_argnames=("interpret",))
def _ohem(preds, target, interpret=False):
    tflat = target.reshape(N_IMG * H, W)
    if interpret:
        hist = jnp.zeros((NW, BINS_W), jnp.float32).at[0, :N_CLS].set(
            jnp.bincount(target.reshape(-1), length=N_CLS).astype(jnp.float32))
    else:
        hist = _classhist_sc(target.reshape(-1))
    hist = hist.reshape(NW * 16, LANE_BINS)
    out = pl.pallas_call(
        _ohem_body,
        grid=(N_IMG, N_HB),
        in_specs=[
            pl.BlockSpec((1, N_CLS, HB, W), lambda n, h: (n, 0, h, 0)),
            pl.BlockSpec((HB, W), lambda n, h: (n * N_HB + h, 0)),
            pl.BlockSpec((NW * 16, LANE_BINS), lambda n, h: (0, 0)),
        ],
        out_specs=pl.BlockSpec((1, 1), lambda n, h: (0, 0)),
        out_shape=jax.ShapeDtypeStruct((1, 1), jnp.float32),
        scratch_shapes=[
            pltpu.VMEM((N_IMG * H, W), jnp.float32),
            pltpu.SMEM((N_CLS,), jnp.float32),
        ],
        interpret=interpret,
    )(preds, tflat, hist)
    return out[0, 0]


def kernel(preds, target):
    return _ohem(preds, target)


# speedup vs baseline: 1.4857x; 1.4857x over previous
import functools

import jax
import jax.numpy as jnp
from jax.experimental import pallas as pl
from jax.experimental.pallas import tpu as pltpu

N_IMG, N_CLS, H, W = 4, 19, 512, 512
N_PIX = N_IMG * H * W
K_HARD = max(100000, int(N_PIX * 0.7))
HB = 64
N_HB = H // HB
BISECT_ITERS = 15


def _select_tree(bits, leaves):
    level = list(leaves)
    for b in bits:
        if len(level) == 1:
            break
        nxt = []
        for j in range(0, len(level) - 1, 2):
            nxt.append(jnp.where(b, level[j + 1], level[j]))
        if len(level) % 2:
            nxt.append(level[-1])
        level = nxt
    return level[0]


def _ohem_body(p_ref, t_ref, tfull_ref, out_ref, loss_buf, w_sm):
    n = pl.program_id(0)
    h = pl.program_id(1)

    @pl.when((n == 0) & (h == 0))
    def _():
        tf = tfull_ref[...]
        for c in range(N_CLS):
            cnt = jnp.sum((tf == c).astype(jnp.float32))
            w_sm[c] = 2.0 - cnt * (1.0 / N_PIX)

    p = p_ref[0]
    t = t_ref[...]
    s = jnp.zeros((HB, W), jnp.float32)
    for c in range(N_CLS):
        s = s + jnp.exp(p[c])
    bits = [((t >> k) & 1) != 0 for k in range(5)]
    pt = _select_tree(bits, [p[c] for c in range(N_CLS)])
    wp = _select_tree(bits, [w_sm[c] for c in range(N_CLS)])
    loss = wp * (jnp.log(s) - pt)
    row = (n * N_HB + h) * HB
    loss_buf[pl.ds(row, HB), :] = loss

    @pl.when((n == N_IMG - 1) & (h == N_HB - 1))
    def _():
        lb = loss_buf[...]
        kf = jnp.float32(K_HARD)

        def it(_, carry):
            lo, hi = carry
            mid = 0.5 * (lo + hi)
            cnt = jnp.sum((lb > mid).astype(jnp.float32))
            take = cnt >= kf
            return jnp.where(take, mid, lo), jnp.where(take, hi, mid)

        lo, hi = jax.lax.fori_loop(
            0, BISECT_ITERS, it, (jnp.float32(0.0), jnp.max(lb)))
        mid = 0.5 * (lo + hi)
        msk = lb > hi
        cnt_gt = jnp.sum(msk.astype(jnp.float32))
        sum_gt = jnp.sum(jnp.where(msk, lb, 0.0))
        hard_sum = sum_gt + (kf - cnt_gt) * mid
        loss_val = hard_sum * (1.0 / (H * W)) * (1.0 / N_IMG)
        out_ref[...] = jnp.full((1, 1), loss_val, jnp.float32)


@functools.partial(jax.jit, static_argnames=("interpret",))
def _ohem(preds, target, interpret=False):
    tflat = target.reshape(N_IMG * H, W)
    out = pl.pallas_call(
        _ohem_body,
        grid=(N_IMG, N_HB),
        in_specs=[
            pl.BlockSpec((1, N_CLS, HB, W), lambda n, h: (n, 0, h, 0)),
            pl.BlockSpec((HB, W), lambda n, h: (n * N_HB + h, 0)),
            pl.BlockSpec((N_IMG * H, W), lambda n, h: (0, 0)),
        ],
        out_specs=pl.BlockSpec((1, 1), lambda n, h: (0, 0)),
        out_shape=jax.ShapeDtypeStruct((1, 1), jnp.float32),
        scratch_shapes=[
            pltpu.VMEM((N_IMG * H, W), jnp.float32),
            pltpu.SMEM((N_CLS,), jnp.float32),
        ],
        interpret=interpret,
    )(preds, tflat, tflat)
    return out[0, 0]


def kernel(preds, target):
    return _ohem(preds, target)


# speedup vs baseline: 1.6414x; 1.1048x over previous
import functools

import jax
import jax.numpy as jnp
from jax.experimental import pallas as pl
from jax.experimental.pallas import tpu as pltpu

N_IMG, N_CLS, H, W = 4, 19, 512, 512
N_PIX = N_IMG * H * W
K_HARD = max(100000, int(N_PIX * 0.7))
HB = 64
N_HB = H // HB
SUB_ROWS = 128
SUB_FRAC = SUB_ROWS * W
K_SUB = (K_HARD * SUB_FRAC) // N_PIX
SUB_ITERS = 18
REFINE_ITERS = 6


def _select_tree(bits, leaves):
    level = list(leaves)
    for b in bits:
        if len(level) == 1:
            break
        nxt = []
        for j in range(0, len(level) - 1, 2):
            nxt.append(jnp.where(b, level[j + 1], level[j]))
        if len(level) % 2:
            nxt.append(level[-1])
        level = nxt
    return level[0]


def _ohem_body(p_ref, t_ref, tfull_ref, out_ref, loss_buf, w_sm):
    n = pl.program_id(0)
    h = pl.program_id(1)

    @pl.when((n == 0) & (h == 0))
    def _():
        tf = tfull_ref[...]
        for c in range(N_CLS):
            cnt = jnp.sum((tf == c).astype(jnp.float32))
            w_sm[c] = 2.0 - cnt * (1.0 / N_PIX)

    p = p_ref[0]
    t = t_ref[...]
    s = jnp.zeros((HB, W), jnp.float32)
    for c in range(N_CLS):
        s = s + jnp.exp(p[c])
    bits = [((t >> k) & 1) != 0 for k in range(5)]
    pt = _select_tree(bits, [p[c] for c in range(N_CLS)])
    wp = _select_tree(bits, [w_sm[c] for c in range(N_CLS)])
    loss = wp * (jnp.log(s) - pt)
    row = (n * N_HB + h) * HB
    loss_buf[pl.ds(row, HB), :] = loss

    @pl.when((n == N_IMG - 1) & (h == N_HB - 1))
    def _():
        lb = loss_buf[...]
        sub = loss_buf[0:SUB_ROWS, :]
        kf = jnp.float32(K_HARD)
        kf_sub = jnp.float32(K_SUB)

        def cnt_gt(x, thr):
            return jnp.sum((x > thr).astype(jnp.float32))

        def it_sub(_, carry):
            lo, hi = carry
            mid = 0.5 * (lo + hi)
            take = cnt_gt(sub, mid) >= kf_sub
            return jnp.where(take, mid, lo), jnp.where(take, hi, mid)

        lo_s, hi_s = jax.lax.fori_loop(
            0, SUB_ITERS, it_sub, (jnp.float32(0.0), jnp.max(sub) + 1.0))

        def bad(carry):
            lo, hi = carry
            return (cnt_gt(lb, lo) < kf) | (cnt_gt(lb, hi) >= kf)

        def widen(carry):
            lo, hi = carry
            span = jnp.maximum(hi - lo, jnp.float32(1e-3))
            return jnp.maximum(lo - 2.0 * span, 0.0) - 1e-6, hi + 2.0 * span

        lo, hi = jax.lax.while_loop(
            bad, widen, (lo_s * 0.97 - 1e-6, hi_s * 1.03 + 1e-6))

        def it_full(_, carry):
            lo, hi = carry
            mid = 0.5 * (lo + hi)
            take = cnt_gt(lb, mid) >= kf
            return jnp.where(take, mid, lo), jnp.where(take, hi, mid)

        lo, hi = jax.lax.fori_loop(0, REFINE_ITERS, it_full, (lo, hi))
        mid = 0.5 * (lo + hi)
        msk = lb > hi
        cnt_gt = jnp.sum(msk.astype(jnp.float32))
        sum_gt = jnp.sum(jnp.where(msk, lb, 0.0))
        hard_sum = sum_gt + (kf - cnt_gt) * mid
        loss_val = hard_sum * (1.0 / (H * W)) * (1.0 / N_IMG)
        out_ref[...] = jnp.full((1, 1), loss_val, jnp.float32)


@functools.partial(jax.jit, static_argnames=("interpret",))
def _ohem(preds, target, interpret=False):
    tflat = target.reshape(N_IMG * H, W)
    out = pl.pallas_call(
        _ohem_body,
        grid=(N_IMG, N_HB),
        in_specs=[
            pl.BlockSpec((1, N_CLS, HB, W), lambda n, h: (n, 0, h, 0)),
            pl.BlockSpec((HB, W), lambda n, h: (n * N_HB + h, 0)),
            pl.BlockSpec((N_IMG * H, W), lambda n, h: (0, 0)),
        ],
        out_specs=pl.BlockSpec((1, 1), lambda n, h: (0, 0)),
        out_shape=jax.ShapeDtypeStruct((1, 1), jnp.float32),
        scratch_shapes=[
            pltpu.VMEM((N_IMG * H, W), jnp.float32),
            pltpu.SMEM((N_CLS,), jnp.float32),
        ],
        interpret=interpret,
    )(preds, tflat, tflat)
    return out[0, 0]


def kernel(preds, target):
    return _ohem(preds, target)


# speedup vs baseline: 1.9525x; 1.1895x over previous
import functools

import jax
import jax.numpy as jnp
from jax.experimental import pallas as pl
from jax.experimental.pallas import tpu as pltpu

N_IMG, N_CLS, H, W = 4, 19, 512, 512
N_PIX = N_IMG * H * W
K_HARD = max(100000, int(N_PIX * 0.7))
HB = 128
N_HB = H // HB
SUB_ROWS = 128
SUB_FRAC = SUB_ROWS * W
K_SUB = (K_HARD * SUB_FRAC) // N_PIX
SUB_ITERS = 18
REFINE_ITERS = 5


def _select_tree(bits, leaves):
    level = list(leaves)
    for b in bits:
        if len(level) == 1:
            break
        nxt = []
        for j in range(0, len(level) - 1, 2):
            nxt.append(jnp.where(b, level[j + 1], level[j]))
        if len(level) % 2:
            nxt.append(level[-1])
        level = nxt
    return level[0]


def _ohem_body(p_ref, t_ref, tfull_ref, out_ref, loss_buf, w_sm):
    n = pl.program_id(0)
    h = pl.program_id(1)

    @pl.when((n == 0) & (h == 0))
    def _():
        tf = tfull_ref[...]
        for c in range(N_CLS):
            cnt = jnp.sum((tf == c).astype(jnp.float32))
            w_sm[c] = 2.0 - cnt * (1.0 / N_PIX)

    p = p_ref[0]
    t = t_ref[...]
    s = jnp.zeros((HB, W), jnp.float32)
    for c in range(N_CLS):
        s = s + jnp.exp(p[c])
    bits = [((t >> k) & 1) != 0 for k in range(5)]
    pt = _select_tree(bits, [p[c] for c in range(N_CLS)])
    wp = _select_tree(bits, [w_sm[c] for c in range(N_CLS)])
    loss = wp * (jnp.log(s) - pt)
    row = (n * N_HB + h) * HB
    loss_buf[pl.ds(row, HB), :] = loss

    @pl.when((n == N_IMG - 1) & (h == N_HB - 1))
    def _():
        lb = loss_buf[...]
        sub = loss_buf[0:SUB_ROWS, :]
        kf = jnp.float32(K_HARD)
        kf_sub = jnp.float32(K_SUB)

        def cnt_gt(x, thr):
            return jnp.sum((x > thr).astype(jnp.float32))

        def it_sub(_, carry):
            lo, hi = carry
            mid = 0.5 * (lo + hi)
            take = cnt_gt(sub, mid) >= kf_sub
            return jnp.where(take, mid, lo), jnp.where(take, hi, mid)

        lo_s, hi_s = jax.lax.fori_loop(
            0, SUB_ITERS, it_sub, (jnp.float32(0.0), jnp.max(sub) + 1.0))

        def bad(carry):
            lo, hi = carry
            return (cnt_gt(lb, lo) < kf) | (cnt_gt(lb, hi) >= kf)

        def widen(carry):
            lo, hi = carry
            span = jnp.maximum(hi - lo, jnp.float32(1e-3))
            return jnp.maximum(lo - 2.0 * span, 0.0) - 1e-6, hi + 2.0 * span

        lo, hi = jax.lax.while_loop(
            bad, widen, (lo_s * 0.97 - 1e-6, hi_s * 1.03 + 1e-6))

        def it_full(_, carry):
            lo, hi = carry
            mid = 0.5 * (lo + hi)
            take = cnt_gt(lb, mid) >= kf
            return jnp.where(take, mid, lo), jnp.where(take, hi, mid)

        lo, hi = jax.lax.fori_loop(0, REFINE_ITERS, it_full, (lo, hi))
        mid = 0.5 * (lo + hi)
        msk = lb > hi
        cnt_gt = jnp.sum(msk.astype(jnp.float32))
        sum_gt = jnp.sum(jnp.where(msk, lb, 0.0))
        hard_sum = sum_gt + (kf - cnt_gt) * mid
        loss_val = hard_sum * (1.0 / (H * W)) * (1.0 / N_IMG)
        out_ref[...] = jnp.full((1, 1), loss_val, jnp.float32)


@functools.partial(jax.jit, static_argnames=("interpret",))
def _ohem(preds, target, interpret=False):
    tflat = target.reshape(N_IMG * H, W)
    out = pl.pallas_call(
        _ohem_body,
        grid=(N_IMG, N_HB),
        in_specs=[
            pl.BlockSpec((1, N_CLS, HB, W), lambda n, h: (n, 0, h, 0)),
            pl.BlockSpec((HB, W), lambda n, h: (n * N_HB + h, 0)),
            pl.BlockSpec((N_IMG * H, W), lambda n, h: (0, 0)),
        ],
        out_specs=pl.BlockSpec((1, 1), lambda n, h: (0, 0)),
        out_shape=jax.ShapeDtypeStruct((1, 1), jnp.float32),
        scratch_shapes=[
            pltpu.VMEM((N_IMG * H, W), jnp.float32),
            pltpu.SMEM((N_CLS,), jnp.float32),
        ],
        interpret=interpret,
    )(preds, tflat, tflat)
    return out[0, 0]


def kernel(preds, target):
    return _ohem(preds, target)


# speedup vs baseline: 2.0633x; 1.0567x over previous
import functools

import jax
import jax.numpy as jnp
from jax.experimental import pallas as pl
from jax.experimental.pallas import tpu as pltpu

N_IMG, N_CLS, H, W = 4, 19, 512, 512
N_PIX = N_IMG * H * W
K_HARD = max(100000, int(N_PIX * 0.7))
HB = 256
N_HB = H // HB
SUB_ROWS = 128
SUB_FRAC = SUB_ROWS * W
K_SUB = (K_HARD * SUB_FRAC) // N_PIX
SUB_ITERS = 18
REFINE_ITERS = 5


def _select_tree(bits, leaves):
    level = list(leaves)
    for b in bits:
        if len(level) == 1:
            break
        nxt = []
        for j in range(0, len(level) - 1, 2):
            nxt.append(jnp.where(b, level[j + 1], level[j]))
        if len(level) % 2:
            nxt.append(level[-1])
        level = nxt
    return level[0]


def _ohem_body(p_ref, t_ref, tfull_ref, out_ref, loss_buf, w_sm):
    n = pl.program_id(0)
    h = pl.program_id(1)

    @pl.when((n == 0) & (h == 0))
    def _():
        tf = tfull_ref[...]
        for c in range(N_CLS):
            cnt = jnp.sum((tf == c).astype(jnp.float32))
            w_sm[c] = 2.0 - cnt * (1.0 / N_PIX)

    p = p_ref[0]
    t = t_ref[...]
    s = jnp.zeros((HB, W), jnp.float32)
    for c in range(N_CLS):
        s = s + jnp.exp(p[c])
    bits = [((t >> k) & 1) != 0 for k in range(5)]
    pt = _select_tree(bits, [p[c] for c in range(N_CLS)])
    wp = _select_tree(bits, [w_sm[c] for c in range(N_CLS)])
    loss = wp * (jnp.log(s) - pt)
    row = (n * N_HB + h) * HB
    loss_buf[pl.ds(row, HB), :] = loss

    @pl.when((n == N_IMG - 1) & (h == N_HB - 1))
    def _():
        lb = loss_buf[...]
        sub = loss_buf[0:SUB_ROWS, :]
        kf = jnp.float32(K_HARD)
        kf_sub = jnp.float32(K_SUB)

        def cnt_gt(x, thr):
            return jnp.sum((x > thr).astype(jnp.float32))

        def it_sub(_, carry):
            lo, hi = carry
            mid = 0.5 * (lo + hi)
            take = cnt_gt(sub, mid) >= kf_sub
            return jnp.where(take, mid, lo), jnp.where(take, hi, mid)

        lo_s, hi_s = jax.lax.fori_loop(
            0, SUB_ITERS, it_sub, (jnp.float32(0.0), jnp.max(sub) + 1.0))

        def bad(carry):
            lo, hi = carry
            return (cnt_gt(lb, lo) < kf) | (cnt_gt(lb, hi) >= kf)

        def widen(carry):
            lo, hi = carry
            span = jnp.maximum(hi - lo, jnp.float32(1e-3))
            return jnp.maximum(lo - 2.0 * span, 0.0) - 1e-6, hi + 2.0 * span

        lo, hi = jax.lax.while_loop(
            bad, widen, (lo_s * 0.97 - 1e-6, hi_s * 1.03 + 1e-6))

        def it_full(_, carry):
            lo, hi = carry
            mid = 0.5 * (lo + hi)
            take = cnt_gt(lb, mid) >= kf
            return jnp.where(take, mid, lo), jnp.where(take, hi, mid)

        lo, hi = jax.lax.fori_loop(0, REFINE_ITERS, it_full, (lo, hi))
        mid = 0.5 * (lo + hi)
        msk = lb > hi
        cnt_gt = jnp.sum(msk.astype(jnp.float32))
        sum_gt = jnp.sum(jnp.where(msk, lb, 0.0))
        hard_sum = sum_gt + (kf - cnt_gt) * mid
        loss_val = hard_sum * (1.0 / (H * W)) * (1.0 / N_IMG)
        out_ref[...] = jnp.full((1, 1), loss_val, jnp.float32)


@functools.partial(jax.jit, static_argnames=("interpret",))
def _ohem(preds, target, interpret=False):
    tflat = target.reshape(N_IMG * H, W)
    out = pl.pallas_call(
        _ohem_body,
        grid=(N_IMG, N_HB),
        in_specs=[
            pl.BlockSpec((1, N_CLS, HB, W), lambda n, h: (n, 0, h, 0)),
            pl.BlockSpec((HB, W), lambda n, h: (n * N_HB + h, 0)),
            pl.BlockSpec((N_IMG * H, W), lambda n, h: (0, 0)),
        ],
        out_specs=pl.BlockSpec((1, 1), lambda n, h: (0, 0)),
        out_shape=jax.ShapeDtypeStruct((1, 1), jnp.float32),
        scratch_shapes=[
            pltpu.VMEM((N_IMG * H, W), jnp.float32),
            pltpu.SMEM((N_CLS,), jnp.float32),
        ],
        interpret=interpret,
    )(preds, tflat, tflat)
    return out[0, 0]


def kernel(preds, target):
    return _ohem(preds, target)
